# Initial kernel scaffold; baseline (speedup 1.0000x reference)
import functools

import jax
import jax.numpy as jnp
from jax import lax
from jax.experimental import pallas as pl
from jax.experimental.pallas import tpu as pltpu
from jax.experimental.pallas import tpu_sc as plsc

NUM_FIELDS = 26
VOCAB = 100000
EMB = 32
BATCH = 16384

N_ROWS = BATCH * NUM_FIELDS
NC, NS, LANES = 2, 16, 16
NW = NC * NS
ROWS_PER_W = N_ROWS // NW
CHUNK = 1664
NCHUNKS = ROWS_PER_W // CHUNK
GATHER = 128
NGATHER = CHUNK // GATHER
NVEC = CHUNK // LANES
NROWBLK = N_ROWS // CHUNK


def _sc_gather(tab, x2d, offs):
    mesh = plsc.VectorSubcoreMesh(core_axis_name="c", subcore_axis_name="s")

    @functools.partial(
        pl.kernel,
        mesh=mesh,
        out_type=jax.ShapeDtypeStruct((NROWBLK, CHUNK, EMB), jnp.float32),
        scratch_types=[
            pltpu.VMEM((CHUNK,), jnp.int32),
            pltpu.VMEM((CHUNK,), jnp.int32),
            pltpu.VMEM((CHUNK,), jnp.int32),
            pltpu.VMEM((CHUNK, EMB), jnp.float32),
            pltpu.SemaphoreType.DMA,
        ],
    )
    def k(x_hbm, offs_hbm, tab_hbm, out_hbm, x_v, offs_v, idx_v, rows_v, sem):
        wid = lax.axis_index("s") * NC + lax.axis_index("c")
        pltpu.sync_copy(offs_hbm, offs_v)

        def chunk_body(ci, carry):
            blk = wid * NCHUNKS + ci
            pltpu.sync_copy(x_hbm.at[blk], x_v)

            def add_body(i, c2):
                o = pl.multiple_of(i * LANES, LANES)
                idx_v[pl.ds(o, LANES)] = (
                    x_v[pl.ds(o, LANES)] + offs_v[pl.ds(o, LANES)]
                )
                return c2

            lax.fori_loop(0, NVEC, add_body, 0)

            cps = [
                pltpu.async_copy(
                    tab_hbm.at[idx_v.at[pl.ds(g * GATHER, GATHER)]],
                    rows_v.at[pl.ds(g * GATHER, GATHER)],
                    sem,
                )
                for g in range(NGATHER)
            ]
            for cp in cps:
                cp.wait()
            pltpu.sync_copy(rows_v, out_hbm.at[blk])
            return carry

        lax.fori_loop(0, NCHUNKS, chunk_body, 0)

    return k(x2d, offs, tab)


def kernel(x, tables):
    xf = x.astype(jnp.int32).reshape(NROWBLK, CHUNK)
    tab = tables.reshape(NUM_FIELDS * VOCAB, EMB)
    offs = (jnp.arange(CHUNK, dtype=jnp.int32) % NUM_FIELDS) * VOCAB
    out = _sc_gather(tab, xf, offs)
    return out.reshape(BATCH, NUM_FIELDS * EMB)


# speedup vs baseline: 1.2056x; 1.2056x over previous
import functools

import jax
import jax.numpy as jnp
from jax import lax
from jax.experimental import pallas as pl
from jax.experimental.pallas import tpu as pltpu
from jax.experimental.pallas import tpu_sc as plsc

NUM_FIELDS = 26
VOCAB = 100000
EMB = 32
BATCH = 16384

N_ROWS = BATCH * NUM_FIELDS
NC, NS, LANES = 2, 16, 16
NW = NC * NS
ROWS_PER_W = N_ROWS // NW
CHUNK = 1664
NCHUNKS = ROWS_PER_W // CHUNK
GATHER = 128
NGATHER = CHUNK // GATHER
NVEC = CHUNK // LANES
NROWBLK = N_ROWS // CHUNK


def _sc_gather(tab, x2d, offs):
    mesh = plsc.VectorSubcoreMesh(core_axis_name="c", subcore_axis_name="s")

    @functools.partial(
        pl.kernel,
        mesh=mesh,
        out_type=jax.ShapeDtypeStruct((NROWBLK, CHUNK, EMB), jnp.float32),
        scratch_types=[
            pltpu.VMEM((CHUNK,), jnp.int32),
            pltpu.VMEM((CHUNK,), jnp.int32),
            pltpu.VMEM((CHUNK,), jnp.int32),
            pltpu.VMEM((CHUNK, EMB), jnp.float32),
            pltpu.SemaphoreType.DMA,
        ],
        compiler_params=pltpu.CompilerParams(use_tc_tiling_on_sc=False),
    )
    def k(x_hbm, offs_hbm, tab_hbm, out_hbm, x_v, offs_v, idx_v, rows_v, sem):
        wid = lax.axis_index("s") * NC + lax.axis_index("c")
        pltpu.sync_copy(offs_hbm, offs_v)

        def chunk_body(ci, carry):
            blk = wid * NCHUNKS + ci
            pltpu.sync_copy(x_hbm.at[blk], x_v)

            def add_body(i, c2):
                o = pl.multiple_of(i * LANES, LANES)
                idx_v[pl.ds(o, LANES)] = (
                    x_v[pl.ds(o, LANES)] + offs_v[pl.ds(o, LANES)]
                )
                return c2

            lax.fori_loop(0, NVEC, add_body, 0)

            cps = [
                pltpu.async_copy(
                    tab_hbm.at[idx_v.at[pl.ds(g * GATHER, GATHER)]],
                    rows_v.at[pl.ds(g * GATHER, GATHER)],
                    sem,
                )
                for g in range(NGATHER)
            ]
            for cp in cps:
                cp.wait()
            pltpu.sync_copy(rows_v, out_hbm.at[blk])
            return carry

        lax.fori_loop(0, NCHUNKS, chunk_body, 0)

    return k(x2d, offs, tab)


def kernel(x, tables):
    xf = x.astype(jnp.int32).reshape(NROWBLK, CHUNK)
    tab = tables.reshape(NUM_FIELDS * VOCAB, EMB)
    offs = (jnp.arange(CHUNK, dtype=jnp.int32) % NUM_FIELDS) * VOCAB
    out = _sc_gather(tab, xf, offs)
    return out.reshape(BATCH, NUM_FIELDS * EMB)


# speedup vs baseline: 1.5048x; 1.2481x over previous
import functools

import jax
import jax.numpy as jnp
from jax import lax
from jax.experimental import pallas as pl
from jax.experimental.pallas import tpu as pltpu
from jax.experimental.pallas import tpu_sc as plsc

NUM_FIELDS = 26
VOCAB = 100000
EMB = 32
BATCH = 16384

N_ROWS = BATCH * NUM_FIELDS
NC, NS, LANES = 2, 16, 16
NW = NC * NS
ROWS_PER_W = N_ROWS // NW
CHUNK = 1664
NCHUNKS = ROWS_PER_W // CHUNK
GATHER = 128
NGATHER = CHUNK // GATHER
NVEC = CHUNK // LANES
NROWBLK = N_ROWS // CHUNK

PACK = 128 // EMB


def _tc_relayout(tab_t):
    QUART = VOCAB // PACK
    STEP = 1000

    def body(in_ref, out_ref):
        for s in range(QUART // STEP):
            for j in range(PACK):
                blk = in_ref[0, :, j * QUART + s * STEP:j * QUART + (s + 1) * STEP]
                out_ref[s * STEP:(s + 1) * STEP, j * EMB:(j + 1) * EMB] = blk.T

    return pl.pallas_call(
        body,
        grid=(NUM_FIELDS,),
        in_specs=[
            pl.BlockSpec((1, EMB, VOCAB), lambda f: (f, 0, 0)),
        ],
        out_specs=pl.BlockSpec((VOCAB // PACK, 128), lambda f: (f, 0)),
        out_shape=jax.ShapeDtypeStruct(
            (NUM_FIELDS * VOCAB // PACK, 128), jnp.float32
        ),
    )(tab_t)


def _sc_gather(tab, x2d, offs):
    mesh = plsc.VectorSubcoreMesh(core_axis_name="c", subcore_axis_name="s")

    @functools.partial(
        pl.kernel,
        mesh=mesh,
        out_type=jax.ShapeDtypeStruct((NROWBLK, CHUNK, EMB), jnp.float32),
        scratch_types=[
            pltpu.VMEM((CHUNK,), jnp.int32),
            pltpu.VMEM((CHUNK,), jnp.int32),
            pltpu.VMEM((CHUNK,), jnp.int32),
            pltpu.VMEM((CHUNK, EMB), jnp.float32),
            pltpu.SemaphoreType.DMA,
        ],
        compiler_params=pltpu.CompilerParams(use_tc_tiling_on_sc=False),
    )
    def k(x_hbm, offs_hbm, tab_hbm, out_hbm, x_v, offs_v, idx_v, rows_v, sem):
        wid = lax.axis_index("s") * NC + lax.axis_index("c")
        pltpu.sync_copy(offs_hbm, offs_v)

        def chunk_body(ci, carry):
            blk = wid * NCHUNKS + ci
            pltpu.sync_copy(x_hbm.at[blk], x_v)

            def add_body(i, c2):
                o = pl.multiple_of(i * LANES, LANES)
                v = x_v[pl.ds(o, LANES)]
                quart = (
                    jnp.where(v >= (VOCAB // PACK), 1, 0)
                    + jnp.where(v >= 2 * (VOCAB // PACK), 1, 0)
                    + jnp.where(v >= 3 * (VOCAB // PACK), 1, 0)
                )
                idx_v[pl.ds(o, LANES)] = (
                    offs_v[pl.ds(o, LANES)]
                    + (v - quart * (VOCAB // PACK)) * PACK
                    + quart
                )
                return c2

            lax.fori_loop(0, NVEC, add_body, 0)

            cps = [
                pltpu.async_copy(
                    tab_hbm.at[idx_v.at[pl.ds(g * GATHER, GATHER)]],
                    rows_v.at[pl.ds(g * GATHER, GATHER)],
                    sem,
                )
                for g in range(NGATHER)
            ]
            for cp in cps:
                cp.wait()
            pltpu.sync_copy(rows_v, out_hbm.at[blk])
            return carry

        lax.fori_loop(0, NCHUNKS, chunk_body, 0)

    return k(x2d, offs, tab)


def kernel(x, tables):
    tab_flat = _tc_relayout(tables.transpose(0, 2, 1)).reshape(
        NUM_FIELDS * VOCAB, EMB
    )
    xf = x.astype(jnp.int32).reshape(NROWBLK, CHUNK)
    offs = (jnp.arange(CHUNK, dtype=jnp.int32) % NUM_FIELDS) * VOCAB
    out = _sc_gather(tab_flat, xf, offs)
    return out.reshape(BATCH, NUM_FIELDS * EMB)


# speedup vs baseline: 3.6130x; 2.4010x over previous
import functools

import jax
import jax.numpy as jnp
from jax import lax
from jax.experimental import pallas as pl
from jax.experimental.pallas import tpu as pltpu
from jax.experimental.pallas import tpu_sc as plsc

NUM_FIELDS = 26
VOCAB = 100000
EMB = 32
BATCH = 16384

N_ROWS = BATCH * NUM_FIELDS
NC, NS, LANES = 2, 16, 16
NW = NC * NS
ROWS_PER_W = N_ROWS // NW
CHUNK = 1664
NCHUNKS = ROWS_PER_W // CHUNK
GATHER = 128
NGATHER = CHUNK // GATHER
NVEC = CHUNK // LANES
NROWBLK = N_ROWS // CHUNK

PACK = 128 // EMB


def _tc_relayout(tab_t):
    QUART = VOCAB // PACK
    STEP = 1000

    def body(in_ref, out_ref):
        for s in range(QUART // STEP):
            m = jnp.concatenate(
                [
                    in_ref[0, :, j * QUART + s * STEP:j * QUART + (s + 1) * STEP]
                    for j in range(PACK)
                ],
                axis=0,
            )
            out_ref[s * STEP:(s + 1) * STEP, :] = m.T

    return pl.pallas_call(
        body,
        grid=(NUM_FIELDS,),
        in_specs=[
            pl.BlockSpec((1, EMB, VOCAB), lambda f: (f, 0, 0)),
        ],
        out_specs=pl.BlockSpec((VOCAB // PACK, 128), lambda f: (f, 0)),
        out_shape=jax.ShapeDtypeStruct(
            (NUM_FIELDS * VOCAB // PACK, 128), jnp.float32
        ),
        compiler_params=pltpu.CompilerParams(vmem_limit_bytes=100 * 2**20),
    )(tab_t)


def _sc_gather(tab, x2d, offs):
    mesh = plsc.VectorSubcoreMesh(core_axis_name="c", subcore_axis_name="s")

    @functools.partial(
        pl.kernel,
        mesh=mesh,
        out_type=jax.ShapeDtypeStruct((NROWBLK, CHUNK, EMB), jnp.float32),
        scratch_types=[
            pltpu.VMEM((CHUNK,), jnp.int32),
            pltpu.VMEM((CHUNK,), jnp.int32),
            pltpu.VMEM((CHUNK,), jnp.int32),
            pltpu.VMEM((CHUNK, EMB), jnp.float32),
            pltpu.SemaphoreType.DMA,
        ],
        compiler_params=pltpu.CompilerParams(use_tc_tiling_on_sc=False),
    )
    def k(x_hbm, offs_hbm, tab_hbm, out_hbm, x_v, offs_v, idx_v, rows_v, sem):
        wid = lax.axis_index("s") * NC + lax.axis_index("c")
        pltpu.sync_copy(offs_hbm, offs_v)

        def chunk_body(ci, carry):
            blk = wid * NCHUNKS + ci
            pltpu.sync_copy(x_hbm.at[blk], x_v)

            def add_body(i, c2):
                o = pl.multiple_of(i * LANES, LANES)
                v = x_v[pl.ds(o, LANES)]
                quart = (
                    jnp.where(v >= (VOCAB // PACK), 1, 0)
                    + jnp.where(v >= 2 * (VOCAB // PACK), 1, 0)
                    + jnp.where(v >= 3 * (VOCAB // PACK), 1, 0)
                )
                idx_v[pl.ds(o, LANES)] = (
                    offs_v[pl.ds(o, LANES)]
                    + (v - quart * (VOCAB // PACK)) * PACK
                    + quart
                )
                return c2

            lax.fori_loop(0, NVEC, add_body, 0)

            cps = [
                pltpu.async_copy(
                    tab_hbm.at[idx_v.at[pl.ds(g * GATHER, GATHER)]],
                    rows_v.at[pl.ds(g * GATHER, GATHER)],
                    sem,
                )
                for g in range(NGATHER)
            ]
            for cp in cps:
                cp.wait()
            pltpu.sync_copy(rows_v, out_hbm.at[blk])
            return carry

        lax.fori_loop(0, NCHUNKS, chunk_body, 0)

    return k(x2d, offs, tab)


def kernel(x, tables):
    tab_flat = _tc_relayout(tables.transpose(0, 2, 1)).reshape(
        NUM_FIELDS * VOCAB, EMB
    )
    xf = x.astype(jnp.int32).reshape(NROWBLK, CHUNK)
    offs = (jnp.arange(CHUNK, dtype=jnp.int32) % NUM_FIELDS) * VOCAB
    out = _sc_gather(tab_flat, xf, offs)
    return out.reshape(BATCH, NUM_FIELDS * EMB)


# speedup vs baseline: 3.7754x; 1.0449x over previous
import functools

import jax
import jax.numpy as jnp
from jax import lax
from jax.experimental import pallas as pl
from jax.experimental.pallas import tpu as pltpu
from jax.experimental.pallas import tpu_sc as plsc

NUM_FIELDS = 26
VOCAB = 100000
EMB = 32
BATCH = 16384

N_ROWS = BATCH * NUM_FIELDS
NC, NS, LANES = 2, 16, 16
NW = NC * NS
ROWS_PER_W = N_ROWS // NW
CHUNK = 1664
NCHUNKS = ROWS_PER_W // CHUNK
GATHER = 128
NGATHER = CHUNK // GATHER
NVEC = CHUNK // LANES
NROWBLK = N_ROWS // CHUNK

PACK = 128 // EMB


def _tc_relayout(tab_t):
    QUART = VOCAB // PACK
    STEP = 5000

    def body(in_ref, out_ref):
        for s in range(QUART // STEP):
            m = jnp.concatenate(
                [
                    in_ref[0, :, j * QUART + s * STEP:j * QUART + (s + 1) * STEP]
                    for j in range(PACK)
                ],
                axis=0,
            )
            out_ref[s * STEP:(s + 1) * STEP, :] = m.T

    return pl.pallas_call(
        body,
        grid=(NUM_FIELDS,),
        in_specs=[
            pl.BlockSpec((1, EMB, VOCAB), lambda f: (f, 0, 0)),
        ],
        out_specs=pl.BlockSpec((VOCAB // PACK, 128), lambda f: (f, 0)),
        out_shape=jax.ShapeDtypeStruct(
            (NUM_FIELDS * VOCAB // PACK, 128), jnp.float32
        ),
        compiler_params=pltpu.CompilerParams(vmem_limit_bytes=100 * 2**20),
    )(tab_t)


def _sc_gather(tab, x2d, offs):
    mesh = plsc.VectorSubcoreMesh(core_axis_name="c", subcore_axis_name="s")

    @functools.partial(
        pl.kernel,
        mesh=mesh,
        out_type=jax.ShapeDtypeStruct((NROWBLK, CHUNK, EMB), jnp.float32),
        scratch_types=[
            pltpu.VMEM((CHUNK,), jnp.int32),
            pltpu.VMEM((CHUNK,), jnp.int32),
            pltpu.VMEM((CHUNK,), jnp.int32),
            pltpu.VMEM((CHUNK, EMB), jnp.float32),
            pltpu.SemaphoreType.DMA,
        ],
        compiler_params=pltpu.CompilerParams(use_tc_tiling_on_sc=False),
    )
    def k(x_hbm, offs_hbm, tab_hbm, out_hbm, x_v, offs_v, idx_v, rows_v, sem):
        wid = lax.axis_index("s") * NC + lax.axis_index("c")
        pltpu.sync_copy(offs_hbm, offs_v)

        def chunk_body(ci, carry):
            blk = wid * NCHUNKS + ci
            pltpu.sync_copy(x_hbm.at[blk], x_v)

            def add_body(i, c2):
                o = pl.multiple_of(i * LANES, LANES)
                v = x_v[pl.ds(o, LANES)]
                quart = (
                    jnp.where(v >= (VOCAB // PACK), 1, 0)
                    + jnp.where(v >= 2 * (VOCAB // PACK), 1, 0)
                    + jnp.where(v >= 3 * (VOCAB // PACK), 1, 0)
                )
                idx_v[pl.ds(o, LANES)] = (
                    offs_v[pl.ds(o, LANES)]
                    + (v - quart * (VOCAB // PACK)) * PACK
                    + quart
                )
                return c2

            lax.fori_loop(0, NVEC, add_body, 0)

            cps = [
                pltpu.async_copy(
                    tab_hbm.at[idx_v.at[pl.ds(g * GATHER, GATHER)]],
                    rows_v.at[pl.ds(g * GATHER, GATHER)],
                    sem,
                )
                for g in range(NGATHER)
            ]
            for cp in cps:
                cp.wait()
            pltpu.sync_copy(rows_v, out_hbm.at[blk])
            return carry

        lax.fori_loop(0, NCHUNKS, chunk_body, 0)

    return k(x2d, offs, tab)


def kernel(x, tables):
    tab_flat = _tc_relayout(tables.transpose(0, 2, 1)).reshape(
        NUM_FIELDS * VOCAB, EMB
    )
    xf = x.astype(jnp.int32).reshape(NROWBLK, CHUNK)
    offs = (jnp.arange(CHUNK, dtype=jnp.int32) % NUM_FIELDS) * VOCAB
    out = _sc_gather(tab_flat, xf, offs)
    return out.reshape(BATCH, NUM_FIELDS * EMB)
